# baseline (speedup 1.0000x reference)
import functools

import jax
import jax.numpy as jnp
from jax import lax
from jax.experimental import pallas as pl
from jax.experimental.pallas import tpu as pltpu
from jax.experimental.pallas import tpu_sc as plsc

N_FIELDS = 26
VOCAB = 100000
EMBED = 32
N_NUM = 13
N_TOK = N_FIELDS + N_NUM
LANES = 16
GROUPS = (4, 7, 7, 8)


def _run_group(xcat_t, xnum_t, tblg, w, bvec, *, batch, nf, with_num):
    info = plsc.get_sparse_core_info()
    nc, ns = info.num_cores, info.num_subcores
    nw = nc * ns
    assert nw == EMBED, "one worker per embed lane"
    ntok = nf + (N_NUM if with_num else 0)
    qb = batch // 4
    nblk = qb // LANES
    unroll = 8

    mesh = plsc.VectorSubcoreMesh(core_axis_name="c", subcore_axis_name="s")

    def body(xcat_ref, xnum_ref, tbl_ref, w_ref, b_ref, out_ref,
             plane_v, idx_v, oq_v, w_v, b_v, wsem):
        e = lax.axis_index("s") * nc + lax.axis_index("c")

        pltpu.sync_copy(w_ref, w_v)
        pltpu.sync_copy(b_ref, b_v)
        ee = jnp.full((LANES,), e, jnp.int32)
        we = plsc.load_gather(w_v, [ee])
        be = plsc.load_gather(b_v, [ee])

        def plane_body(t, carry):
            is_cat = t < nf

            @pl.when(is_cat)
            def _():
                pltpu.sync_copy(xcat_ref.at[t], idx_v)
                pltpu.sync_copy(tbl_ref.at[t, e], plane_v)

            @pl.when(jnp.logical_not(is_cat))
            def _():
                pltpu.sync_copy(xnum_ref.at[t - nf],
                                plane_v.at[pl.ds(0, batch)])

            def q_body(q, c):
                qbuf = oq_v.at[q % 2]

                @pl.when(t * 4 + q >= 2)
                def _():
                    pltpu.make_async_copy(
                        qbuf, out_ref.at[t, e, pl.ds(0, qb)], wsem).wait()

                @pl.when(is_cat)
                def _():
                    def blk(k, cc):
                        for u in range(unroll):
                            o = k * unroll + u
                            sl = pl.ds(q * qb + o * LANES, LANES)
                            idx16 = idx_v[sl]
                            s = plsc.load_gather(plane_v, [idx16])
                            qbuf[pl.ds(o * LANES, LANES)] = s
                        return cc

                    lax.fori_loop(0, nblk // unroll, blk, 0)

                @pl.when(jnp.logical_not(is_cat))
                def _():
                    def blk(k, cc):
                        for u in range(unroll):
                            o = k * unroll + u
                            sl = pl.ds(q * qb + o * LANES, LANES)
                            v16 = plane_v[sl]
                            qbuf[pl.ds(o * LANES, LANES)] = v16 * we + be
                        return cc

                    lax.fori_loop(0, nblk // unroll, blk, 0)

                pltpu.async_copy(qbuf, out_ref.at[t, e, pl.ds(q * qb, qb)],
                                 wsem)
                return c

            lax.fori_loop(0, 4, q_body, carry)
            return carry

        lax.fori_loop(0, ntok, plane_body, 0)

        def final_drain(q, c):
            pltpu.make_async_copy(
                oq_v.at[q], out_ref.at[ntok - 1, e, pl.ds(0, qb)],
                wsem).wait()
            return c

        lax.fori_loop(0, 2, final_drain, 0)

    call = pl.kernel(
        body,
        out_type=jax.ShapeDtypeStruct((ntok, EMBED, batch), jnp.float32),
        mesh=mesh,
        scratch_types=[
            pltpu.VMEM((VOCAB,), jnp.float32),
            pltpu.VMEM((batch,), jnp.int32),
            pltpu.VMEM((2, qb), jnp.float32),
            pltpu.VMEM((EMBED,), jnp.float32),
            pltpu.VMEM((EMBED,), jnp.float32),
            pltpu.SemaphoreType.DMA,
        ],
        compiler_params=pltpu.CompilerParams(
            use_tc_tiling_on_sc=False, needs_layout_passes=False),
    )
    return call(xcat_t, xnum_t, tblg, w, bvec)


def _run_num(xnum_t, w, bvec, *, batch):
    info = plsc.get_sparse_core_info()
    nc, ns = info.num_cores, info.num_subcores
    assert nc * ns == EMBED
    qb = batch // 4
    nblk = qb // LANES
    unroll = 8

    mesh = plsc.VectorSubcoreMesh(core_axis_name="c", subcore_axis_name="s")

    def body(xnum_ref, w_ref, b_ref, out_ref, col_v, oq_v, w_v, b_v, wsem):
        e = lax.axis_index("s") * nc + lax.axis_index("c")
        pltpu.sync_copy(w_ref, w_v)
        pltpu.sync_copy(b_ref, b_v)
        ee = jnp.full((LANES,), e, jnp.int32)
        we = plsc.load_gather(w_v, [ee])
        be = plsc.load_gather(b_v, [ee])

        def plane_body(t, carry):
            pltpu.sync_copy(xnum_ref.at[t], col_v)

            def q_body(q, c):
                qbuf = oq_v.at[q % 2]

                @pl.when(t * 4 + q >= 2)
                def _():
                    pltpu.make_async_copy(
                        qbuf, out_ref.at[t, e, pl.ds(0, qb)], wsem).wait()

                def blk(k, cc):
                    for u in range(unroll):
                        o = k * unroll + u
                        v16 = col_v[pl.ds(q * qb + o * LANES, LANES)]
                        qbuf[pl.ds(o * LANES, LANES)] = v16 * we + be
                    return cc

                lax.fori_loop(0, nblk // unroll, blk, 0)
                pltpu.async_copy(qbuf, out_ref.at[t, e, pl.ds(q * qb, qb)],
                                 wsem)
                return c

            lax.fori_loop(0, 4, q_body, carry)
            return carry

        lax.fori_loop(0, N_NUM, plane_body, 0)

        def final_drain(q, c):
            pltpu.make_async_copy(
                oq_v.at[q], out_ref.at[N_NUM - 1, e, pl.ds(0, qb)],
                wsem).wait()
            return c

        lax.fori_loop(0, 2, final_drain, 0)

    call = pl.kernel(
        body,
        out_type=jax.ShapeDtypeStruct((N_NUM, EMBED, batch), jnp.float32),
        mesh=mesh,
        scratch_types=[
            pltpu.VMEM((batch,), jnp.float32),
            pltpu.VMEM((2, qb), jnp.float32),
            pltpu.VMEM((EMBED,), jnp.float32),
            pltpu.VMEM((EMBED,), jnp.float32),
            pltpu.SemaphoreType.DMA,
        ],
        compiler_params=pltpu.CompilerParams(
            use_tc_tiling_on_sc=False, needs_layout_passes=False),
    )
    return call(xnum_t, w, bvec)


@functools.partial(jax.jit, static_argnames=("batch",))
def _run(x_cat, x_num, tables, W, b, *, batch):
    xnum_t = x_num.T
    w = W.reshape(EMBED)
    outs = []
    f0 = 0
    for nf in GROUPS:
        xcat_g = x_cat[:, f0:f0 + nf].astype(jnp.int32).T
        tblg = jnp.transpose(tables[f0:f0 + nf], (0, 2, 1))
        outs.append(
            _run_group(xcat_g, xnum_t, tblg, w, b,
                       batch=batch, nf=nf, with_num=False))
        f0 += nf
    outs.append(_run_num(xnum_t, w, b, batch=batch))
    out_t = jnp.concatenate(outs, axis=0)
    return jnp.transpose(out_t, (2, 0, 1))


def kernel(x_cat, x_num, tables, W, b):
    return _run(x_cat, x_num, tables, W, b, batch=x_cat.shape[0])


# speedup vs baseline: 1.0599x; 1.0599x over previous
import functools

import jax
import jax.numpy as jnp
from jax import lax
from jax.experimental import pallas as pl
from jax.experimental.pallas import tpu as pltpu
from jax.experimental.pallas import tpu_sc as plsc

N_FIELDS = 26
VOCAB = 100000
EMBED = 32
N_NUM = 13
N_TOK = N_FIELDS + N_NUM
LANES = 16
GROUPS = (4, 7, 7, 8)


def _run_group(xcat_t, xnum_t, tblg, w, bvec, *, batch, nf, with_num):
    info = plsc.get_sparse_core_info()
    nc, ns = info.num_cores, info.num_subcores
    nw = nc * ns
    assert nw == EMBED, "one worker per embed lane"
    ntok = nf + (N_NUM if with_num else 0)
    qb = batch // 4
    nblk = qb // LANES
    unroll = 8

    mesh = plsc.VectorSubcoreMesh(core_axis_name="c", subcore_axis_name="s")

    def body(xcat_ref, xnum_ref, tbl_ref, w_ref, b_ref, out_ref,
             plane_v, idx_v, oq_v, w_v, b_v, wsem):
        e = lax.axis_index("s") * nc + lax.axis_index("c")

        pltpu.sync_copy(w_ref, w_v)
        pltpu.sync_copy(b_ref, b_v)
        ee = jnp.full((LANES,), e, jnp.int32)
        we = plsc.load_gather(w_v, [ee])
        be = plsc.load_gather(b_v, [ee])

        def plane_body(t, carry):
            is_cat = t < nf

            @pl.when(is_cat)
            def _():
                pltpu.sync_copy(xcat_ref.at[t], idx_v)
                pltpu.sync_copy(tbl_ref.at[t, e], plane_v)

            @pl.when(jnp.logical_not(is_cat))
            def _():
                pltpu.sync_copy(xnum_ref.at[t - nf],
                                plane_v.at[pl.ds(0, batch)])

            def q_body(q, c):
                qbuf = oq_v.at[q % 2]

                @pl.when(t * 4 + q >= 2)
                def _():
                    pltpu.make_async_copy(
                        qbuf, out_ref.at[t, e, pl.ds(0, qb)], wsem).wait()

                @pl.when(is_cat)
                def _():
                    def blk(k, cc):
                        for u in range(unroll):
                            o = k * unroll + u
                            sl = pl.ds(q * qb + o * LANES, LANES)
                            idx16 = idx_v[sl]
                            s = plsc.load_gather(plane_v, [idx16])
                            qbuf[pl.ds(o * LANES, LANES)] = s
                        return cc

                    lax.fori_loop(0, nblk // unroll, blk, 0)

                @pl.when(jnp.logical_not(is_cat))
                def _():
                    def blk(k, cc):
                        for u in range(unroll):
                            o = k * unroll + u
                            sl = pl.ds(q * qb + o * LANES, LANES)
                            v16 = plane_v[sl]
                            qbuf[pl.ds(o * LANES, LANES)] = v16 * we + be
                        return cc

                    lax.fori_loop(0, nblk // unroll, blk, 0)

                pltpu.async_copy(qbuf, out_ref.at[t, e, pl.ds(q * qb, qb)],
                                 wsem)
                return c

            lax.fori_loop(0, 4, q_body, carry)
            return carry

        lax.fori_loop(0, ntok, plane_body, 0)

        def final_drain(q, c):
            pltpu.make_async_copy(
                oq_v.at[q], out_ref.at[ntok - 1, e, pl.ds(0, qb)],
                wsem).wait()
            return c

        lax.fori_loop(0, 2, final_drain, 0)

    call = pl.kernel(
        body,
        out_type=jax.ShapeDtypeStruct((ntok, EMBED, batch), jnp.float32),
        mesh=mesh,
        scratch_types=[
            pltpu.VMEM((VOCAB,), jnp.float32),
            pltpu.VMEM((batch,), jnp.int32),
            pltpu.VMEM((2, qb), jnp.float32),
            pltpu.VMEM((EMBED,), jnp.float32),
            pltpu.VMEM((EMBED,), jnp.float32),
            pltpu.SemaphoreType.DMA,
        ],
        compiler_params=pltpu.CompilerParams(
            use_tc_tiling_on_sc=False, needs_layout_passes=False),
    )
    return call(xcat_t, xnum_t, tblg, w, bvec)


@functools.partial(jax.jit, static_argnames=("batch",))
def _run(x_cat, x_num, tables, W, b, *, batch):
    xnum_t = x_num.T
    w = W.reshape(EMBED)
    outs = []
    f0 = 0
    for gi, nf in enumerate(GROUPS):
        last = gi == len(GROUPS) - 1
        xcat_g = x_cat[:, f0:f0 + nf].astype(jnp.int32).T
        tblg = jnp.transpose(tables[f0:f0 + nf], (0, 2, 1))
        outs.append(
            _run_group(xcat_g, xnum_t, tblg, w, b,
                       batch=batch, nf=nf, with_num=last))
        f0 += nf
    out_t = jnp.concatenate(outs, axis=0)
    return jnp.transpose(out_t, (2, 0, 1))


def kernel(x_cat, x_num, tables, W, b):
    return _run(x_cat, x_num, tables, W, b, batch=x_cat.shape[0])
